# baseline (speedup 1.0000x reference)
import functools

import jax
import jax.numpy as jnp
from jax import lax
from jax.experimental import pallas as pl
from jax.experimental.pallas import tpu as pltpu
from jax.experimental.pallas import tpu_sc as plsc

BATCH = 4096
HIST = 200
EMBED_DIM = 64

NUM_CORES = 2
NUM_SUBCORES = 16
NUM_WORKERS = NUM_CORES * NUM_SUBCORES
L = 16
BW = BATCH // NUM_WORKERS // 4
BLK = 128

_mesh = plsc.VectorSubcoreMesh(core_axis_name="c", subcore_axis_name="s")


@functools.partial(
    pl.kernel,
    mesh=_mesh,
    out_type=jax.ShapeDtypeStruct((HIST, 8, BATCH // BLK, 8, BLK),
                                  jnp.float32),
    scratch_types=[
        [pltpu.VMEM((BLK,), jnp.int32) for _ in range(2)],
        [pltpu.VMEM((BLK, EMBED_DIM), jnp.float32) for _ in range(2)],
        [pltpu.VMEM((8, 8, BLK), jnp.float32) for _ in range(2)],
        [pltpu.SemaphoreType.DMA for _ in range(2)],
        [pltpu.SemaphoreType.DMA for _ in range(2)],
        [pltpu.SemaphoreType.DMA for _ in range(2)],
    ],
    compiler_params=pltpu.CompilerParams(
        use_tc_tiling_on_sc=False, needs_layout_passes=False),
)
def _sc_gather(xT_hbm, table_hbm, out_hbm, idxb, rows, outb,
               xsems, gsems, osems):
    w = lax.axis_index("s") * NUM_CORES + lax.axis_index("c")
    b0 = w * BLK
    iotas = [lax.iota(jnp.int32, L) + L * k for k in range(8)]

    def idx_start(h, p):
        pltpu.async_copy(xT_hbm.at[h, pl.ds(b0, BLK)], idxb[p], xsems[p])

    def idx_wait(p):
        pltpu.make_async_copy(
            xT_hbm.at[0, pl.ds(b0, BLK)], idxb[p], xsems[p]).wait()

    def gather_start(p):
        pltpu.async_copy(table_hbm.at[idxb[p]], rows[p], gsems[p])

    def gather_wait(p):
        pltpu.make_async_copy(
            table_hbm.at[idxb[p]], rows[p], gsems[p]).wait()

    def wb_start(h, p):
        pltpu.async_copy(outb[p], out_hbm.at[h, :, w, :, :], osems[p])

    def wb_wait(p):
        pltpu.make_async_copy(
            outb[p], out_hbm.at[0, :, w, :, :], osems[p]).wait()

    def transpose(p):
        for e in range(EMBED_DIM):
            ev = lax.broadcast(jnp.int32(e), (L,))
            for k in range(8):
                vals = plsc.load_gather(rows[p], [iotas[k], ev])
                outb[p][e // 8, e % 8, pl.ds(L * k, L)] = vals

    for p in range(2):
        idx_start(p, p)
        idx_wait(p)
        gather_start(p)

    def body(g, carry):
        for p in range(2):
            h = 2 * g + p
            gather_wait(p)

            @pl.when(h + 2 < HIST)
            def _():
                idx_start(h + 2, p)

            @pl.when(g > 0)
            def _():
                wb_wait(p)

            transpose(p)
            wb_start(h, p)

            @pl.when(h + 2 < HIST)
            def _():
                idx_wait(p)
                gather_start(p)

        return carry

    lax.fori_loop(0, HIST // 2, body, 0)

    for p in range(2):
        wb_wait(p)


def kernel(x, table):
    out5 = _sc_gather(x.T, table)
    return jnp.transpose(out5, (2, 4, 0, 1, 3)).reshape(
        x.shape + (table.shape[1],))


# speedup vs baseline: 1.9867x; 1.9867x over previous
import functools

import jax
import jax.numpy as jnp
from jax import lax
from jax.experimental import pallas as pl
from jax.experimental.pallas import tpu as pltpu
from jax.experimental.pallas import tpu_sc as plsc

BATCH = 4096
HIST = 200
EMBED_DIM = 64

NUM_CORES = 2
NUM_SUBCORES = 16
NUM_WORKERS = NUM_CORES * NUM_SUBCORES
L = 16
BW = BATCH // NUM_WORKERS // 4
BLK = 128

_mesh = plsc.VectorSubcoreMesh(core_axis_name="c", subcore_axis_name="s")


@functools.partial(
    pl.kernel,
    mesh=_mesh,
    out_type=jax.ShapeDtypeStruct((HIST, 8, BATCH // BLK, 8, BLK),
                                  jnp.float32),
    scratch_types=[
        [pltpu.VMEM((BLK,), jnp.int32) for _ in range(2)],
        [pltpu.VMEM((BLK, EMBED_DIM), jnp.float32) for _ in range(2)],
        [pltpu.VMEM((8, 8, BLK), jnp.float32) for _ in range(2)],
        [pltpu.SemaphoreType.DMA for _ in range(2)],
        [pltpu.SemaphoreType.DMA for _ in range(2)],
        [pltpu.SemaphoreType.DMA for _ in range(2)],
    ],
    compiler_params=pltpu.CompilerParams(
        use_tc_tiling_on_sc=False, needs_layout_passes=False),
)
def _sc_gather(xT_hbm, table_hbm, out_hbm, idxb, rows, outb,
               xsems, gsems, osems):
    w = lax.axis_index("s") * NUM_CORES + lax.axis_index("c")
    b0 = w * BLK
    iotas = [lax.iota(jnp.int32, L) + L * k for k in range(8)]

    def idx_start(h, p):
        pltpu.async_copy(xT_hbm.at[h, pl.ds(b0, BLK)], idxb[p], xsems[p])

    def idx_wait(p):
        pltpu.make_async_copy(
            xT_hbm.at[0, pl.ds(b0, BLK)], idxb[p], xsems[p]).wait()

    def gather_start(p):
        pltpu.async_copy(table_hbm.at[idxb[p]], rows[p], gsems[p])

    def gather_wait(p):
        pltpu.make_async_copy(
            table_hbm.at[idxb[p]], rows[p], gsems[p]).wait()

    def wb_start(h, p):
        pltpu.async_copy(outb[p], out_hbm.at[h, :, w, :, :], osems[p])

    def wb_wait(p):
        pltpu.make_async_copy(
            outb[p], out_hbm.at[0, :, w, :, :], osems[p]).wait()

    def transpose(p):
        def body(e, carry):
            r = (e + iotas[0]) & 63
            te = r >> 3
            ee = r & 7
            for k in range(8):
                vals = plsc.load_gather(rows[p], [iotas[k], r])
                plsc.store_scatter(outb[p], [te, ee, iotas[k]], vals)
            return carry

        lax.fori_loop(0, EMBED_DIM, body, 0)

    for p in range(2):
        idx_start(p, p)
        idx_wait(p)
        gather_start(p)

    def body(g, carry):
        for p in range(2):
            h = 2 * g + p
            gather_wait(p)

            @pl.when(h + 2 < HIST)
            def _():
                idx_start(h + 2, p)

            @pl.when(g > 0)
            def _():
                wb_wait(p)

            transpose(p)
            wb_start(h, p)

            @pl.when(h + 2 < HIST)
            def _():
                idx_wait(p)
                gather_start(p)

        return carry

    lax.fori_loop(0, HIST // 2, body, 0)

    for p in range(2):
        wb_wait(p)


def kernel(x, table):
    out5 = _sc_gather(x.T, table)
    return jnp.transpose(out5, (2, 4, 0, 1, 3)).reshape(
        x.shape + (table.shape[1],))


# speedup vs baseline: 2.1856x; 1.1001x over previous
import functools

import jax
import jax.numpy as jnp
from jax import lax
from jax.experimental import pallas as pl
from jax.experimental.pallas import tpu as pltpu
from jax.experimental.pallas import tpu_sc as plsc

BATCH = 4096
HIST = 200
EMBED_DIM = 64
SLOT = 128

NUM_CORES = 2
NUM_SUBCORES = 16
NUM_WORKERS = NUM_CORES * NUM_SUBCORES

N = BATCH * HIST
PER_WORKER = N // NUM_WORKERS
CHUNK = 256
NBUF = 4
NUM_CHUNKS = PER_WORKER // CHUNK
NUM_GROUPS = NUM_CHUNKS // NBUF

_mesh = plsc.VectorSubcoreMesh(core_axis_name="c", subcore_axis_name="s")


@functools.partial(
    pl.kernel,
    mesh=_mesh,
    out_type=jax.ShapeDtypeStruct((N, SLOT), jnp.float32),
    scratch_types=[
        pltpu.VMEM((PER_WORKER,), jnp.int32),
        [pltpu.VMEM((CHUNK, EMBED_DIM), jnp.float32) for _ in range(NBUF)],
        [pltpu.SemaphoreType.DMA for _ in range(NBUF)],
        [pltpu.SemaphoreType.DMA for _ in range(NBUF)],
    ],
    compiler_params=pltpu.CompilerParams(use_tc_tiling_on_sc=False),
)
def _sc_gather(idx_hbm, table_hbm, out_hbm, idx_v, rows, gsems, osems):
    wid = lax.axis_index("s") * NUM_CORES + lax.axis_index("c")
    base = wid * PER_WORKER

    pltpu.sync_copy(idx_hbm.at[pl.ds(base, PER_WORKER)], idx_v)

    def gather_start(chunk_i, k):
        pltpu.async_copy(
            table_hbm.at[idx_v.at[pl.ds(chunk_i * CHUNK, CHUNK)]],
            rows[k], gsems[k])

    def gather_wait(k):
        pltpu.make_async_copy(
            table_hbm.at[idx_v.at[pl.ds(0, CHUNK)]],
            rows[k], gsems[k]).wait()

    def wb_start(chunk_i, k):
        pltpu.async_copy(
            rows[k],
            out_hbm.at[pl.ds(base + chunk_i * CHUNK, CHUNK),
                       pl.ds(0, EMBED_DIM)],
            osems[k])

    def wb_wait(k):
        pltpu.make_async_copy(
            rows[k],
            out_hbm.at[pl.ds(base, CHUNK), pl.ds(0, EMBED_DIM)],
            osems[k]).wait()

    for k in range(NBUF):
        gather_start(k, k)

    def body(g, carry):
        c0 = g * NBUF
        for k in range(NBUF):
            gather_wait(k)
            wb_start(c0 + k, k)

        @pl.when(g < NUM_GROUPS - 1)
        def _():
            for k in range(NBUF):
                wb_wait(k)
                gather_start(c0 + NBUF + k, k)

        return carry

    lax.fori_loop(0, NUM_GROUPS, body, 0)

    for k in range(NBUF):
        wb_wait(k)


def kernel(x, table):
    idx = x.reshape(-1).astype(jnp.int32)
    out128 = _sc_gather(idx, table)
    return out128[:, :EMBED_DIM].reshape(x.shape + (table.shape[1],))
